# baseline (speedup 1.0000x reference)
import functools
import jax
import jax.numpy as jnp
from jax import lax
from jax.experimental import pallas as pl
from jax.experimental.pallas import tpu as pltpu
from jax.experimental.pallas import tpu_sc as plsc

H = 32
ROW = 48
NC = 2
NS = 16
LANES = 16
NW = NC * NS
CH = 80
NBUF = 5
LOOKAHEAD = 3


def _leaky(x, slope):
    return jnp.where(x >= 0, x, slope * x)


def _node_stage(hwg, att_ref, hw_ref, tbl_ref):
    hw_ref[...] = hwg
    t = hwg[:, 0:H] @ att_ref[...]
    z = t[:, 0:1] + t[:, 1:2]
    b = _leaky(z, 0.2)
    ci = lax.broadcasted_iota(jnp.int32, t.shape, 1)
    tbl_ref[...] = jnp.where(ci == 2, jnp.broadcast_to(b, t.shape), t)


def _tc0_body(x_ref, win_ref, bin_ref, w0p_ref, e32_ref, att_ref,
              hw_ref, tbl_ref):
    act = _leaky(x_ref[...] @ win_ref[...] + bin_ref[...], 0.01)
    _node_stage(act @ w0p_ref[...] + e32_ref[...], att_ref, hw_ref, tbl_ref)


def _tc_mid_body(parts_ref, bc_ref, wp_ref, e32_ref, att_ref,
                 hw_ref, tbl_ref):
    tot = parts_ref[0] + parts_ref[1]
    act = _leaky(tot[:, 0:H] / (tot[:, H:H + 1] + 1e-16) + bc_ref[...], 0.01)
    _node_stage(act @ wp_ref[...] + e32_ref[...], att_ref, hw_ref, tbl_ref)


def _tc_fin_body(parts_ref, bc_ref, wout_ref, bout_ref, o_ref):
    tot = parts_ref[0] + parts_ref[1]
    act = _leaky(tot[:, 0:H] / (tot[:, H:H + 1] + 1e-16) + bc_ref[...], 0.01)
    logits = act @ wout_ref[...] + bout_ref[...]
    m = jnp.max(logits, axis=-1, keepdims=True)
    s = jnp.log(jnp.sum(jnp.exp(logits - m), axis=-1, keepdims=True))
    o_ref[...] = logits - m - s


def _make_sc_layer(n, nchunk):
    assert nchunk % NBUF == 0, "ring pipeline unrolls chunk groups of NBUF"
    nexp = 10
    npt = n // nexp
    assert n % nexp == 0 and npt % 8 == 0
    mesh = plsc.VectorSubcoreMesh(core_axis_name="c", subcore_axis_name="s")

    @functools.partial(
        pl.kernel,
        out_type=jax.ShapeDtypeStruct((NC, n, ROW), jnp.float32),
        mesh=mesh,
        compiler_params=pltpu.CompilerParams(
            needs_layout_passes=False, use_tc_tiling_on_sc=False),
        scratch_types=[
            pltpu.VMEM((nchunk, CH), jnp.int32),
            pltpu.VMEM((nchunk, CH), jnp.int32),
            pltpu.VMEM((4 * n,), jnp.float32),
            pltpu.VMEM((nchunk * CH,), jnp.float32),
            [pltpu.VMEM((CH, ROW), jnp.float32)] * NBUF,
            [pltpu.SemaphoreType.DMA] * NBUF,
            [pltpu.SemaphoreType.DMA] * NBUF,
            pltpu.VMEM_SHARED((n, ROW), jnp.float32),
        ],
    )
    def sc_layer(src_hbm, dst_hbm, hw_hbm, tbl_hbm, zero_hbm,
                 rows_hbm,
                 src_v, dst_v, tbl_v, ex_v, rbs, sgs, sss, acc):
        c = lax.axis_index("c")
        s = lax.axis_index("s")
        gwid = c * NS + s
        pltpu.sync_copy(src_hbm.at[gwid], src_v)
        pltpu.sync_copy(dst_hbm.at[gwid], dst_v)
        pltpu.sync_copy(tbl_hbm, tbl_v)
        slab = pl.ds(s * npt, npt)

        @pl.when(jnp.logical_and(c == 0, s < nexp))
        def _():
            pltpu.sync_copy(hw_hbm.at[slab], acc.at[slab])

        @pl.when(jnp.logical_and(c != 0, s < nexp))
        def _():
            pltpu.sync_copy(zero_hbm.at[slab], acc.at[slab])

        plsc.subcore_barrier()

        def start_gather(i, rb, sem):
            pltpu.async_copy(hw_hbm.at[src_v.at[i]], rb, sem)

        def wait_gather(i, rb, sem):
            pltpu.make_async_copy(hw_hbm.at[src_v.at[i]], rb, sem).wait()

        def start_scatter(i, rb, sem):
            pltpu.async_copy(rb, acc.at[dst_v.at[i]], sem, add=True)

        def wait_scatter(i, rb, sem):
            pltpu.make_async_copy(rb, acc.at[dst_v.at[i]], sem).wait()

        for i in range(LOOKAHEAD):
            start_gather(i, rbs[i], sgs[i])

        def p1(i, carry):
            for j in range(CH // LANES):
                sl = pl.ds(j * LANES, LANES)
                s16 = src_v[i, sl] * 4
                d16 = dst_v[i, sl] * 4
                ga = plsc.load_gather(tbl_v, [s16])
                gd = plsc.load_gather(tbl_v, [d16 + 1])
                gb = plsc.load_gather(tbl_v, [d16 + 2])
                zz = ga + gd
                alpha = jnp.where(zz >= 0, zz, 0.2 * zz)
                ex_v[pl.ds(i * CH + j * LANES, LANES)] = jnp.exp(alpha - gb)
            return carry

        lax.fori_loop(0, nchunk, p1, 0)

        onehot = jnp.where(lax.iota(jnp.int32, LANES) == 0, 1.0, 0.0)

        def scale(i, rb):
            base16 = lax.broadcast(i * CH, (LANES,))
            for e in range(CH):
                mv = plsc.load_gather(ex_v, [base16 + e])
                rb[e, pl.ds(0, LANES)] = rb[e, pl.ds(0, LANES)] * mv
                rb[e, pl.ds(LANES, LANES)] = rb[e, pl.ds(LANES, LANES)] * mv
                rb[e, pl.ds(2 * LANES, LANES)] = mv * onehot

        def p2(k, carry):
            for l in range(NBUF):
                i = NBUF * k + l
                wait_gather(i, rbs[l], sgs[l])
                scale(i, rbs[l])
                start_scatter(i, rbs[l], sss[l])
                i2 = i + LOOKAHEAD
                nl = (l + LOOKAHEAD) % NBUF

                @pl.when(i2 < nchunk)
                def _():
                    @pl.when(i2 >= NBUF)
                    def _():
                        wait_scatter(i2 - NBUF, rbs[nl], sss[nl])

                    start_gather(i2, rbs[nl], sgs[nl])

            return carry

        lax.fori_loop(0, nchunk // NBUF, p2, 0)
        for l in range(NBUF):
            wait_scatter(nchunk - NBUF + l, rbs[l], sss[l])

        plsc.subcore_barrier()

        @pl.when(s < nexp)
        def _():
            pltpu.sync_copy(acc.at[slab], rows_hbm.at[c, slab])

    return sc_layer


def kernel(x, edge_index, edge_weight, W_in, b_in, W0, as0, ad0, bc0,
           W1, as1, ad1, bc1, W2, as2, ad2, bc2, W_out, b_out):
    n, _ = x.shape
    e = edge_index.shape[1]
    assert e % NW == 0 and (e // NW) % CH == 0 and n % NS == 0
    nchunk = (e // NW) // CH

    src3 = edge_index[0].reshape(NW, nchunk, CH)
    dst3 = edge_index[1].reshape(NW, nchunk, CH)
    zero48 = jnp.zeros((n, ROW), jnp.float32)
    e32 = (jnp.arange(ROW) == H).astype(jnp.float32)[None, :]

    def pad_w(w):
        return jnp.concatenate([w, jnp.zeros((H, ROW - H), jnp.float32)], 1)

    def att4(a_s, a_d):
        z = jnp.zeros((H,), jnp.float32)
        return jnp.stack([a_s, a_d, z, z], axis=1)

    two_out = [jax.ShapeDtypeStruct((n, ROW), jnp.float32),
               jax.ShapeDtypeStruct((n, 4), jnp.float32)]

    tc0 = pl.pallas_call(_tc0_body, out_shape=two_out)
    tcm = pl.pallas_call(_tc_mid_body, out_shape=two_out)
    tcf = pl.pallas_call(
        _tc_fin_body,
        out_shape=jax.ShapeDtypeStruct((n, W_out.shape[1]), jnp.float32))
    sc = _make_sc_layer(n, nchunk)

    hw, tbl = tc0(x, W_in, b_in.reshape(1, H), pad_w(W0), e32, att4(as0, ad0))
    parts = sc(src3, dst3, hw, tbl.reshape(-1), zero48)
    hw, tbl = tcm(parts, bc0.reshape(1, H), pad_w(W1), e32, att4(as1, ad1))
    parts = sc(src3, dst3, hw, tbl.reshape(-1), zero48)
    hw, tbl = tcm(parts, bc1.reshape(1, H), pad_w(W2), e32, att4(as2, ad2))
    parts = sc(src3, dst3, hw, tbl.reshape(-1), zero48)
    out = tcf(parts, bc2.reshape(1, H), W_out, b_out.reshape(1, -1))
    return out


# speedup vs baseline: 1.4613x; 1.4613x over previous
import functools
import jax
import jax.numpy as jnp
from jax import lax
from jax.experimental import pallas as pl
from jax.experimental.pallas import tpu as pltpu
from jax.experimental.pallas import tpu_sc as plsc

H = 32
ROW = 48
NC = 2
NS = 16
LANES = 16
NW = NC * NS
CH = 80
NBUF = 5
LOOKAHEAD = 3


def _leaky(x, slope):
    return jnp.where(x >= 0, x, slope * x)


def _node_stage(hwg, att_ref, hw_ref, tbl_ref):
    hw_ref[...] = hwg
    t = hwg[:, 0:H] @ att_ref[...]
    z = t[:, 0:1] + t[:, 1:2]
    b = _leaky(z, 0.2)
    ci = lax.broadcasted_iota(jnp.int32, t.shape, 1)
    tbl_ref[...] = jnp.where(ci == 2, jnp.broadcast_to(b, t.shape), t)


def _tc0_body(x_ref, win_ref, bin_ref, w0p_ref, e32_ref, att_ref,
              hw_ref, tbl_ref):
    act = _leaky(x_ref[...] @ win_ref[...] + bin_ref[...], 0.01)
    _node_stage(act @ w0p_ref[...] + e32_ref[...], att_ref, hw_ref, tbl_ref)


def _tc_mid_body(parts_ref, bc_ref, wp_ref, e32_ref, att_ref,
                 hw_ref, tbl_ref):
    tot = parts_ref[0] + parts_ref[1]
    act = _leaky(tot[:, 0:H] / (tot[:, H:H + 1] + 1e-16) + bc_ref[...], 0.01)
    _node_stage(act @ wp_ref[...] + e32_ref[...], att_ref, hw_ref, tbl_ref)


def _tc_fin_body(parts_ref, bc_ref, wout_ref, bout_ref, o_ref):
    tot = parts_ref[0] + parts_ref[1]
    act = _leaky(tot[:, 0:H] / (tot[:, H:H + 1] + 1e-16) + bc_ref[...], 0.01)
    logits = act @ wout_ref[...] + bout_ref[...]
    m = jnp.max(logits, axis=-1, keepdims=True)
    s = jnp.log(jnp.sum(jnp.exp(logits - m), axis=-1, keepdims=True))
    o_ref[...] = logits - m - s


def _make_sc_layer(n, nchunk):
    assert nchunk % NBUF == 0, "ring pipeline unrolls chunk groups of NBUF"
    nexp = 10
    npt = n // nexp
    assert n % nexp == 0 and npt % 8 == 0
    mesh = plsc.VectorSubcoreMesh(core_axis_name="c", subcore_axis_name="s")

    @functools.partial(
        pl.kernel,
        out_type=jax.ShapeDtypeStruct((NC, n, ROW), jnp.float32),
        mesh=mesh,
        compiler_params=pltpu.CompilerParams(
            needs_layout_passes=False, use_tc_tiling_on_sc=False),
        scratch_types=[
            pltpu.VMEM((nchunk, CH), jnp.int32),
            pltpu.VMEM((nchunk, CH), jnp.int32),
            pltpu.VMEM((4 * n,), jnp.float32),
            pltpu.VMEM((nchunk * CH,), jnp.float32),
            [pltpu.VMEM((CH, ROW), jnp.float32)] * NBUF,
            [pltpu.SemaphoreType.DMA] * NBUF,
            [pltpu.SemaphoreType.DMA] * NBUF,
            pltpu.VMEM_SHARED((n, ROW), jnp.float32),
        ],
    )
    def sc_layer(src_hbm, dst_hbm, hw_hbm, tbl_hbm, zero_hbm,
                 rows_hbm,
                 src_v, dst_v, tbl_v, ex_v, rbs, sgs, sss, acc):
        c = lax.axis_index("c")
        s = lax.axis_index("s")
        gwid = c * NS + s
        pltpu.sync_copy(src_hbm.at[gwid], src_v)
        pltpu.sync_copy(dst_hbm.at[gwid], dst_v)
        pltpu.sync_copy(tbl_hbm, tbl_v)
        slab = pl.ds(s * npt, npt)

        @pl.when(jnp.logical_and(c == 0, s < nexp))
        def _():
            pltpu.sync_copy(hw_hbm.at[slab], acc.at[slab])

        @pl.when(jnp.logical_and(c != 0, s < nexp))
        def _():
            pltpu.sync_copy(zero_hbm.at[slab], acc.at[slab])

        plsc.subcore_barrier()

        def start_gather(i, rb, sem):
            pltpu.async_copy(hw_hbm.at[src_v.at[i]], rb, sem)

        def wait_gather(i, rb, sem):
            pltpu.make_async_copy(hw_hbm.at[src_v.at[i]], rb, sem).wait()

        def start_scatter(i, rb, sem):
            pltpu.async_copy(rb, acc.at[dst_v.at[i]], sem, add=True)

        def wait_scatter(i, rb, sem):
            pltpu.make_async_copy(rb, acc.at[dst_v.at[i]], sem).wait()

        for i in range(LOOKAHEAD):
            start_gather(i, rbs[i], sgs[i])

        def p1(i, carry):
            for j in range(CH // LANES):
                sl = pl.ds(j * LANES, LANES)
                s16 = src_v[i, sl] * 4
                d16 = dst_v[i, sl] * 4
                ga = plsc.load_gather(tbl_v, [s16])
                gd = plsc.load_gather(tbl_v, [d16 + 1])
                gb = plsc.load_gather(tbl_v, [d16 + 2])
                zz = ga + gd
                alpha = jnp.where(zz >= 0, zz, 0.2 * zz)
                ex_v[pl.ds(i * CH + j * LANES, LANES)] = jnp.exp(alpha - gb)
            return carry

        lax.fori_loop(0, nchunk, p1, 0)

        onehot = jnp.where(lax.iota(jnp.int32, LANES) == 0, 1.0, 0.0)

        def scale(i, rb):
            for g in range(CH // LANES):
                ev = ex_v[pl.ds(i * CH + g * LANES, LANES)]
                for l in range(LANES):
                    e = g * LANES + l
                    mv = lax.broadcast(ev[l], (LANES,))
                    rb[e, pl.ds(0, LANES)] = rb[e, pl.ds(0, LANES)] * mv
                    rb[e, pl.ds(LANES, LANES)] = (
                        rb[e, pl.ds(LANES, LANES)] * mv)
                    rb[e, pl.ds(2 * LANES, LANES)] = mv * onehot

        def p2(k, carry):
            for l in range(NBUF):
                i = NBUF * k + l
                wait_gather(i, rbs[l], sgs[l])
                scale(i, rbs[l])
                start_scatter(i, rbs[l], sss[l])
                i2 = i + LOOKAHEAD
                nl = (l + LOOKAHEAD) % NBUF

                @pl.when(i2 < nchunk)
                def _():
                    @pl.when(i2 >= NBUF)
                    def _():
                        wait_scatter(i2 - NBUF, rbs[nl], sss[nl])

                    start_gather(i2, rbs[nl], sgs[nl])

            return carry

        lax.fori_loop(0, nchunk // NBUF, p2, 0)
        for l in range(NBUF):
            wait_scatter(nchunk - NBUF + l, rbs[l], sss[l])

        plsc.subcore_barrier()

        @pl.when(s < nexp)
        def _():
            pltpu.sync_copy(acc.at[slab], rows_hbm.at[c, slab])

    return sc_layer


def kernel(x, edge_index, edge_weight, W_in, b_in, W0, as0, ad0, bc0,
           W1, as1, ad1, bc1, W2, as2, ad2, bc2, W_out, b_out):
    n, _ = x.shape
    e = edge_index.shape[1]
    assert e % NW == 0 and (e // NW) % CH == 0 and n % NS == 0
    nchunk = (e // NW) // CH

    src3 = edge_index[0].reshape(NW, nchunk, CH)
    dst3 = edge_index[1].reshape(NW, nchunk, CH)
    zero48 = jnp.zeros((n, ROW), jnp.float32)
    e32 = (jnp.arange(ROW) == H).astype(jnp.float32)[None, :]

    def pad_w(w):
        return jnp.concatenate([w, jnp.zeros((H, ROW - H), jnp.float32)], 1)

    def att4(a_s, a_d):
        z = jnp.zeros((H,), jnp.float32)
        return jnp.stack([a_s, a_d, z, z], axis=1)

    two_out = [jax.ShapeDtypeStruct((n, ROW), jnp.float32),
               jax.ShapeDtypeStruct((n, 4), jnp.float32)]

    tc0 = pl.pallas_call(_tc0_body, out_shape=two_out)
    tcm = pl.pallas_call(_tc_mid_body, out_shape=two_out)
    tcf = pl.pallas_call(
        _tc_fin_body,
        out_shape=jax.ShapeDtypeStruct((n, W_out.shape[1]), jnp.float32))
    sc = _make_sc_layer(n, nchunk)

    hw, tbl = tc0(x, W_in, b_in.reshape(1, H), pad_w(W0), e32, att4(as0, ad0))
    parts = sc(src3, dst3, hw, tbl.reshape(-1), zero48)
    hw, tbl = tcm(parts, bc0.reshape(1, H), pad_w(W1), e32, att4(as1, ad1))
    parts = sc(src3, dst3, hw, tbl.reshape(-1), zero48)
    hw, tbl = tcm(parts, bc1.reshape(1, H), pad_w(W2), e32, att4(as2, ad2))
    parts = sc(src3, dst3, hw, tbl.reshape(-1), zero48)
    out = tcf(parts, bc2.reshape(1, H), W_out, b_out.reshape(1, -1))
    return out


# speedup vs baseline: 1.6075x; 1.1001x over previous
import functools
import jax
import jax.numpy as jnp
from jax import lax
from jax.experimental import pallas as pl
from jax.experimental.pallas import tpu as pltpu
from jax.experimental.pallas import tpu_sc as plsc

H = 32
ROW = 48
NC = 2
NS = 16
LANES = 16
NW = NC * NS
CH = 80
NBUF = 5
LOOKAHEAD = 3


def _leaky(x, slope):
    return jnp.where(x >= 0, x, slope * x)


def _node_stage(hwg, att_ref, hw_ref, tbl_ref):
    hw_ref[...] = hwg
    t = hwg[:, 0:H] @ att_ref[...]
    z = t[:, 0:1] + t[:, 1:2]
    b = _leaky(z, 0.2)
    ci = lax.broadcasted_iota(jnp.int32, t.shape, 1)
    tbl_ref[...] = jnp.where(ci == 2, jnp.broadcast_to(b, t.shape), t)


def _tc0_body(x_ref, win_ref, bin_ref, w0p_ref, e32_ref, att_ref,
              hw_ref, tbl_ref):
    act = _leaky(x_ref[...] @ win_ref[...] + bin_ref[...], 0.01)
    _node_stage(act @ w0p_ref[...] + e32_ref[...], att_ref, hw_ref, tbl_ref)


def _tc_mid_body(parts_ref, bc_ref, wp_ref, e32_ref, att_ref,
                 hw_ref, tbl_ref):
    tot = parts_ref[0] + parts_ref[1]
    act = _leaky(tot[:, 0:H] / (tot[:, H:H + 1] + 1e-16) + bc_ref[...], 0.01)
    _node_stage(act @ wp_ref[...] + e32_ref[...], att_ref, hw_ref, tbl_ref)


def _tc_fin_body(parts_ref, bc_ref, wout_ref, bout_ref, o_ref):
    tot = parts_ref[0] + parts_ref[1]
    act = _leaky(tot[:, 0:H] / (tot[:, H:H + 1] + 1e-16) + bc_ref[...], 0.01)
    logits = act @ wout_ref[...] + bout_ref[...]
    m = jnp.max(logits, axis=-1, keepdims=True)
    s = jnp.log(jnp.sum(jnp.exp(logits - m), axis=-1, keepdims=True))
    o_ref[...] = logits - m - s


def _make_sc_layer(n, nchunk):
    assert nchunk % NBUF == 0, "ring pipeline unrolls chunk groups of NBUF"
    nexp = 10
    npt = n // nexp
    assert n % nexp == 0 and npt % 8 == 0
    mesh = plsc.VectorSubcoreMesh(core_axis_name="c", subcore_axis_name="s")

    @functools.partial(
        pl.kernel,
        out_type=jax.ShapeDtypeStruct((NC, n, ROW), jnp.float32),
        mesh=mesh,
        compiler_params=pltpu.CompilerParams(
            needs_layout_passes=False, use_tc_tiling_on_sc=False),
        scratch_types=[
            pltpu.VMEM((nchunk, CH), jnp.int32),
            pltpu.VMEM((nchunk, CH), jnp.int32),
            pltpu.VMEM((4 * n,), jnp.float32),
            [pltpu.VMEM((CH, ROW), jnp.float32)] * NBUF,
            [pltpu.SemaphoreType.DMA] * NBUF,
            [pltpu.SemaphoreType.DMA] * NBUF,
            pltpu.VMEM_SHARED((n, ROW), jnp.float32),
        ],
    )
    def sc_layer(src_hbm, dst_hbm, hw_hbm, tbl_hbm, zero_hbm,
                 rows_hbm,
                 src_v, dst_v, tbl_v, rbs, sgs, sss, acc):
        c = lax.axis_index("c")
        s = lax.axis_index("s")
        gwid = c * NS + s
        pltpu.sync_copy(src_hbm.at[gwid], src_v)
        pltpu.sync_copy(dst_hbm.at[gwid], dst_v)
        pltpu.sync_copy(tbl_hbm, tbl_v)
        slab = pl.ds(s * npt, npt)

        @pl.when(jnp.logical_and(c == 0, s < nexp))
        def _():
            pltpu.sync_copy(hw_hbm.at[slab], acc.at[slab])

        @pl.when(jnp.logical_and(c != 0, s < nexp))
        def _():
            pltpu.sync_copy(zero_hbm.at[slab], acc.at[slab])

        plsc.subcore_barrier()

        def start_gather(i, rb, sem):
            pltpu.async_copy(hw_hbm.at[src_v.at[i]], rb, sem)

        def wait_gather(i, rb, sem):
            pltpu.make_async_copy(hw_hbm.at[src_v.at[i]], rb, sem).wait()

        def start_scatter(i, rb, sem):
            pltpu.async_copy(rb, acc.at[dst_v.at[i]], sem, add=True)

        def wait_scatter(i, rb, sem):
            pltpu.make_async_copy(rb, acc.at[dst_v.at[i]], sem).wait()

        for i in range(LOOKAHEAD):
            start_gather(i, rbs[i], sgs[i])

        onehot = jnp.where(lax.iota(jnp.int32, LANES) == 0, 1.0, 0.0)

        def edge_weights(i):
            evs = []
            for j in range(CH // LANES):
                sl = pl.ds(j * LANES, LANES)
                s16 = src_v[i, sl] * 4
                d16 = dst_v[i, sl] * 4
                ga = plsc.load_gather(tbl_v, [s16])
                gd = plsc.load_gather(tbl_v, [d16 + 1])
                gb = plsc.load_gather(tbl_v, [d16 + 2])
                zz = ga + gd
                alpha = jnp.where(zz >= 0, zz, 0.2 * zz)
                evs.append(jnp.exp(alpha - gb))
            return evs

        def scale(evs, rb):
            for g in range(CH // LANES):
                ev = evs[g]
                for l in range(LANES):
                    e = g * LANES + l
                    mv = lax.broadcast(ev[l], (LANES,))
                    rb[e, pl.ds(0, LANES)] = rb[e, pl.ds(0, LANES)] * mv
                    rb[e, pl.ds(LANES, LANES)] = (
                        rb[e, pl.ds(LANES, LANES)] * mv)
                    rb[e, pl.ds(2 * LANES, LANES)] = mv * onehot

        def p2(k, carry):
            for l in range(NBUF):
                i = NBUF * k + l
                evs = edge_weights(i)
                wait_gather(i, rbs[l], sgs[l])
                scale(evs, rbs[l])
                start_scatter(i, rbs[l], sss[l])
                i2 = i + LOOKAHEAD
                nl = (l + LOOKAHEAD) % NBUF

                @pl.when(i2 < nchunk)
                def _():
                    @pl.when(i2 >= NBUF)
                    def _():
                        wait_scatter(i2 - NBUF, rbs[nl], sss[nl])

                    start_gather(i2, rbs[nl], sgs[nl])

            return carry

        lax.fori_loop(0, nchunk // NBUF, p2, 0)
        for l in range(NBUF):
            wait_scatter(nchunk - NBUF + l, rbs[l], sss[l])

        plsc.subcore_barrier()

        @pl.when(s < nexp)
        def _():
            pltpu.sync_copy(acc.at[slab], rows_hbm.at[c, slab])

    return sc_layer


def kernel(x, edge_index, edge_weight, W_in, b_in, W0, as0, ad0, bc0,
           W1, as1, ad1, bc1, W2, as2, ad2, bc2, W_out, b_out):
    n, _ = x.shape
    e = edge_index.shape[1]
    assert e % NW == 0 and (e // NW) % CH == 0 and n % NS == 0
    nchunk = (e // NW) // CH

    src3 = edge_index[0].reshape(NW, nchunk, CH)
    dst3 = edge_index[1].reshape(NW, nchunk, CH)
    zero48 = jnp.zeros((n, ROW), jnp.float32)
    e32 = (jnp.arange(ROW) == H).astype(jnp.float32)[None, :]

    def pad_w(w):
        return jnp.concatenate([w, jnp.zeros((H, ROW - H), jnp.float32)], 1)

    def att4(a_s, a_d):
        z = jnp.zeros((H,), jnp.float32)
        return jnp.stack([a_s, a_d, z, z], axis=1)

    two_out = [jax.ShapeDtypeStruct((n, ROW), jnp.float32),
               jax.ShapeDtypeStruct((n, 4), jnp.float32)]

    tc0 = pl.pallas_call(_tc0_body, out_shape=two_out)
    tcm = pl.pallas_call(_tc_mid_body, out_shape=two_out)
    tcf = pl.pallas_call(
        _tc_fin_body,
        out_shape=jax.ShapeDtypeStruct((n, W_out.shape[1]), jnp.float32))
    sc = _make_sc_layer(n, nchunk)

    hw, tbl = tc0(x, W_in, b_in.reshape(1, H), pad_w(W0), e32, att4(as0, ad0))
    parts = sc(src3, dst3, hw, tbl.reshape(-1), zero48)
    hw, tbl = tcm(parts, bc0.reshape(1, H), pad_w(W1), e32, att4(as1, ad1))
    parts = sc(src3, dst3, hw, tbl.reshape(-1), zero48)
    hw, tbl = tcm(parts, bc1.reshape(1, H), pad_w(W2), e32, att4(as2, ad2))
    parts = sc(src3, dst3, hw, tbl.reshape(-1), zero48)
    out = tcf(parts, bc2.reshape(1, H), W_out, b_out.reshape(1, -1))
    return out
